# baseline (speedup 1.0000x reference)
import jax
import jax.numpy as jnp
from jax import lax
from jax.experimental import pallas as pl
from jax.experimental.pallas import tpu as pltpu

N = 4096
H = N // 2
BM = 256
NB = N // BM
BP = 512
BR = 512
BC = 2048
T_P1A = NB
T_FINA = NB + 4
T_MIX = T_FINA + 1
T_FINB = T_MIX + 8
T_UR = T_FINB + 1
T_LO = T_UR + 4
T_END = T_LO + 8


def _mu_rc(t):
    in_mix = (t >= T_MIX) & (t < T_UR)
    in_ur = (t >= T_UR) & (t < T_LO)
    in_lo = t >= T_LO
    r = jnp.where(in_mix, jnp.minimum((t - T_MIX) // 2, 3),
                  jnp.where(in_ur, t - T_UR,
                            jnp.where(in_lo, 4 + (t - T_LO) // 2, 0)))
    c = jnp.where(in_ur, 1, jnp.where(in_lo, (t - T_LO) % 2, 0))
    return r, c


def _fused_kernel(adj_ref, c_ref, x_ref, w1_ref, wp_ref, wc_ref, wb_ref,
                  mu_ref, adj_bf, x1_bf, y_acc, c_bf, cx_s, colsum_s):
    t = pl.program_id(0)

    @pl.when(t < NB)
    def _phase0():
        i = t
        a = adj_ref[...]
        a_bf = a.astype(jnp.bfloat16)
        adj_bf[pl.ds(i * BM, BM), :] = a_bf

        y = jnp.dot(a_bf, x_ref[...], preferred_element_type=jnp.float32)
        x1t = jnp.maximum(
            jnp.dot(y, w1_ref[...], preferred_element_type=jnp.float32), 0.0)
        x1_bf[pl.ds(i * BM, BM), :] = x1t.astype(jnp.bfloat16)

        c = c_ref[...]
        c_bf_blk = c.astype(jnp.bfloat16)
        c_bf[pl.ds(i * BM, BM), :] = c_bf_blk
        xc = x_ref[pl.ds(i * BM, BM), :]
        cx = lax.dot_general(c_bf_blk, xc, (((0,), (0,)), ((), ())),
                             preferred_element_type=jnp.float32)
        ones = jnp.ones((BM, 1), jnp.float32)
        cs = lax.dot_general(c, ones, (((0,), (0,)), ((), ())),
                             preferred_element_type=jnp.float32)

        @pl.when(t == 0)
        def _init():
            cx_s[...] = cx
            colsum_s[...] = cs

        @pl.when(t > 0)
        def _acc():
            cx_s[...] += cx
            colsum_s[...] += cs

    is_p1a = (t >= T_P1A) & (t < T_FINA)
    is_p1b = (t >= T_MIX) & (t < T_FINB) & ((t - T_MIX) % 2 == 1)

    @pl.when(is_p1a | is_p1b)
    def _phase1():
        i = jnp.where(is_p1a, t - T_P1A, 4 + (t - T_MIX) // 2)
        a_bf = adj_bf[pl.ds(i * BP, BP), :]
        y_acc[pl.ds(i * BP, BP), :] = jnp.dot(
            a_bf, x1_bf[...], preferred_element_type=jnp.float32)

    @pl.when((t == T_FINA) | (t == T_FINB))
    def _finalize_half():
        h1 = jnp.maximum(jnp.dot(cx_s[...], wp_ref[...],
                                 preferred_element_type=jnp.float32), 0.0)
        g = jnp.dot(h1, wb_ref[...], preferred_element_type=jnp.float32)
        gs = (g / colsum_s[...]).astype(jnp.bfloat16)
        base = jnp.where(t == T_FINA, 0, H)
        hb = jnp.dot(c_bf[pl.ds(base, H), :], gs,
                     preferred_element_type=jnp.float32)
        yh = y_acc[pl.ds(base, H), :]
        y_acc[pl.ds(base, H), :] = jnp.maximum(
            jnp.dot(yh, wc_ref[...], preferred_element_type=jnp.float32)
            + hb, 0.0)

    is_write = (((t >= T_MIX) & (t < T_FINB) & ((t - T_MIX) % 2 == 0))
                | (t >= T_UR))

    @pl.when(is_write)
    def _decoder():
        r, c = _mu_rc(t)
        zb = y_acc[pl.ds(r * BR, BR), :]
        rhs = y_acc[pl.ds(c * BC, BC), :]
        mu_ref[...] = jnp.maximum(
            lax.dot_general(zb, rhs, (((1,), (1,)), ((), ())),
                            preferred_element_type=jnp.float32), 0.0)


def kernel(x, adj, C, W1, Wp, Wc, Wb):
    B, n, D = x.shape
    K = C.shape[1]
    x_bf = x[0].astype(jnp.bfloat16)

    mu = pl.pallas_call(
        _fused_kernel,
        grid=(T_END,),
        in_specs=[
            pl.BlockSpec((BM, N), lambda t: (jnp.minimum(t, NB - 1), 0)),
            pl.BlockSpec((BM, K), lambda t: (jnp.minimum(t, NB - 1), 0)),
            pl.BlockSpec((N, D), lambda t: (0, 0)),
            pl.BlockSpec((D, D), lambda t: (0, 0)),
            pl.BlockSpec((D, D), lambda t: (0, 0)),
            pl.BlockSpec((D, D), lambda t: (0, 0)),
            pl.BlockSpec((D, D), lambda t: (0, 0)),
        ],
        out_specs=pl.BlockSpec((BR, BC), _mu_rc),
        out_shape=jax.ShapeDtypeStruct((N, N), jnp.float32),
        scratch_shapes=[
            pltpu.VMEM((N, N), jnp.bfloat16),
            pltpu.VMEM((N, 64), jnp.bfloat16),
            pltpu.VMEM((N, 64), jnp.float32),
            pltpu.VMEM((N, K), jnp.bfloat16),
            pltpu.VMEM((K, 64), jnp.float32),
            pltpu.VMEM((K, 1), jnp.float32),
        ],
        compiler_params=pltpu.CompilerParams(
            vmem_limit_bytes=63 * 1024 * 1024),
    )(adj, C, x_bf, W1, Wp, Wc, Wb)

    return (mu.reshape(B, N, N), x)


# speedup vs baseline: 1.0647x; 1.0647x over previous
import jax
import jax.numpy as jnp
from jax import lax
from jax.experimental import pallas as pl
from jax.experimental.pallas import tpu as pltpu

N = 4096
BM = 256
NB = N // BM
BP = 1024
NP = N // BP
BD = 256
ND = N // BD
T_FIN = NB + NP
T_DEC = T_FIN + 1


def _fused_kernel(adj_ref, c_ref, x_ref, w1_ref, wp_ref, wc_ref, wb_ref,
                  mu_ref, adj_bf, x1_bf, y_acc, c_bf, cx_s, colsum_s):
    t = pl.program_id(0)

    @pl.when(t < NB)
    def _phase0():
        i = t
        a = adj_ref[...]
        a_bf = a.astype(jnp.bfloat16)
        adj_bf[pl.ds(i * BM, BM), :] = a_bf

        y = jnp.dot(a_bf, x_ref[...], preferred_element_type=jnp.float32)
        x1t = jnp.maximum(
            jnp.dot(y, w1_ref[...], preferred_element_type=jnp.float32), 0.0)
        x1_bf[pl.ds(i * BM, BM), :] = x1t.astype(jnp.bfloat16)

        c = c_ref[...]
        c_bf_blk = c.astype(jnp.bfloat16)
        c_bf[pl.ds(i * BM, BM), :] = c_bf_blk
        xc = x_ref[pl.ds(i * BM, BM), :]
        cx = lax.dot_general(c_bf_blk, xc, (((0,), (0,)), ((), ())),
                             preferred_element_type=jnp.float32)
        ones = jnp.ones((BM, 1), jnp.float32)
        cs = lax.dot_general(c, ones, (((0,), (0,)), ((), ())),
                             preferred_element_type=jnp.float32)

        @pl.when(t == 0)
        def _init():
            cx_s[...] = cx
            colsum_s[...] = cs

        @pl.when(t > 0)
        def _acc():
            cx_s[...] += cx
            colsum_s[...] += cs

    @pl.when((t >= NB) & (t < T_FIN))
    def _phase1():
        i = t - NB
        a_bf = adj_bf[pl.ds(i * BP, BP), :]
        y_acc[pl.ds(i * BP, BP), :] = jnp.dot(
            a_bf, x1_bf[...], preferred_element_type=jnp.float32)

    @pl.when(t == T_FIN)
    def _finalize():
        h1 = jnp.maximum(jnp.dot(cx_s[...], wp_ref[...],
                                 preferred_element_type=jnp.float32), 0.0)
        g = jnp.dot(h1, wb_ref[...], preferred_element_type=jnp.float32)
        gs = (g / colsum_s[...]).astype(jnp.bfloat16)
        hb = jnp.dot(c_bf[...], gs, preferred_element_type=jnp.float32)
        y_acc[...] = jnp.maximum(
            jnp.dot(y_acc[...], wc_ref[...],
                    preferred_element_type=jnp.float32) + hb, 0.0)

    @pl.when(t > T_FIN)
    def _phase2():
        i = t - T_DEC
        zb = y_acc[pl.ds(i * BD, BD), :]
        mu_ref[...] = jnp.maximum(
            lax.dot_general(zb, y_acc[...], (((1,), (1,)), ((), ())),
                            preferred_element_type=jnp.float32), 0.0)


def kernel(x, adj, C, W1, Wp, Wc, Wb):
    B, n, D = x.shape
    K = C.shape[1]
    x_bf = x[0].astype(jnp.bfloat16)

    mu = pl.pallas_call(
        _fused_kernel,
        grid=(NB + NP + 1 + ND,),
        in_specs=[
            pl.BlockSpec((BM, N), lambda t: (jnp.minimum(t, NB - 1), 0)),
            pl.BlockSpec((BM, K), lambda t: (jnp.minimum(t, NB - 1), 0)),
            pl.BlockSpec((N, D), lambda t: (0, 0)),
            pl.BlockSpec((D, D), lambda t: (0, 0)),
            pl.BlockSpec((D, D), lambda t: (0, 0)),
            pl.BlockSpec((D, D), lambda t: (0, 0)),
            pl.BlockSpec((D, D), lambda t: (0, 0)),
        ],
        out_specs=pl.BlockSpec((BD, N),
                               lambda t: (jnp.maximum(t - T_DEC, 0), 0)),
        out_shape=jax.ShapeDtypeStruct((N, N), jnp.float32),
        scratch_shapes=[
            pltpu.VMEM((N, N), jnp.bfloat16),
            pltpu.VMEM((N, 64), jnp.bfloat16),
            pltpu.VMEM((N, 64), jnp.float32),
            pltpu.VMEM((N, K), jnp.bfloat16),
            pltpu.VMEM((K, 64), jnp.float32),
            pltpu.VMEM((K, 1), jnp.float32),
        ],
        compiler_params=pltpu.CompilerParams(
            vmem_limit_bytes=63 * 1024 * 1024),
    )(adj, C, x_bf, W1, Wp, Wc, Wb)

    return (mu.reshape(B, N, N), x)


# speedup vs baseline: 1.0678x; 1.0028x over previous
import jax
import jax.numpy as jnp
from jax import lax
from jax.experimental import pallas as pl
from jax.experimental.pallas import tpu as pltpu

N = 4096
BM = 256
NB = N // BM
BP = 2048
NP = N // BP
BD = 256
ND = N // BD
T_FIN = NB + NP
T_DEC = T_FIN + 1


def _fused_kernel(adj_ref, c_ref, x_ref, w1_ref, wp_ref, wc_ref, wb_ref,
                  mu_ref, adj_bf, x1_bf, y_acc, c_bf, cx_s, colsum_s):
    t = pl.program_id(0)

    @pl.when(t < NB)
    def _phase0():
        i = t
        a = adj_ref[...]
        a_bf = a.astype(jnp.bfloat16)
        adj_bf[pl.ds(i * BM, BM), :] = a_bf

        y = jnp.dot(a_bf, x_ref[...], preferred_element_type=jnp.float32)
        x1t = jnp.maximum(
            jnp.dot(y, w1_ref[...], preferred_element_type=jnp.float32), 0.0)
        x1_bf[pl.ds(i * BM, BM), :] = x1t.astype(jnp.bfloat16)

        c = c_ref[...]
        c_bf_blk = c.astype(jnp.bfloat16)
        c_bf[pl.ds(i * BM, BM), :] = c_bf_blk
        xc = x_ref[pl.ds(i * BM, BM), :]
        cx = lax.dot_general(c_bf_blk, xc, (((0,), (0,)), ((), ())),
                             preferred_element_type=jnp.float32)
        ones = jnp.ones((BM, 1), jnp.float32)
        cs = lax.dot_general(c, ones, (((0,), (0,)), ((), ())),
                             preferred_element_type=jnp.float32)

        @pl.when(t == 0)
        def _init():
            cx_s[...] = cx
            colsum_s[...] = cs

        @pl.when(t > 0)
        def _acc():
            cx_s[...] += cx
            colsum_s[...] += cs

    @pl.when((t >= NB) & (t < T_FIN))
    def _phase1():
        i = t - NB
        a_bf = adj_bf[pl.ds(i * BP, BP), :]
        y_acc[pl.ds(i * BP, BP), :] = jnp.dot(
            a_bf, x1_bf[...], preferred_element_type=jnp.float32)

    @pl.when(t == T_FIN)
    def _finalize():
        h1 = jnp.maximum(jnp.dot(cx_s[...], wp_ref[...],
                                 preferred_element_type=jnp.float32), 0.0)
        g = jnp.dot(h1, wb_ref[...], preferred_element_type=jnp.float32)
        gs = (g / colsum_s[...]).astype(jnp.bfloat16)
        hb = jnp.dot(c_bf[...], gs, preferred_element_type=jnp.float32)
        y_acc[...] = jnp.maximum(
            jnp.dot(y_acc[...], wc_ref[...],
                    preferred_element_type=jnp.float32) + hb, 0.0)

    @pl.when(t > T_FIN)
    def _phase2():
        i = t - T_DEC
        zb = y_acc[pl.ds(i * BD, BD), :]
        mu_ref[...] = jnp.maximum(
            lax.dot_general(zb, y_acc[...], (((1,), (1,)), ((), ())),
                            preferred_element_type=jnp.float32), 0.0)


def kernel(x, adj, C, W1, Wp, Wc, Wb):
    B, n, D = x.shape
    K = C.shape[1]
    x_bf = x[0].astype(jnp.bfloat16)

    mu = pl.pallas_call(
        _fused_kernel,
        grid=(NB + NP + 1 + ND,),
        in_specs=[
            pl.BlockSpec((BM, N), lambda t: (jnp.minimum(t, NB - 1), 0)),
            pl.BlockSpec((BM, K), lambda t: (jnp.minimum(t, NB - 1), 0)),
            pl.BlockSpec((N, D), lambda t: (0, 0)),
            pl.BlockSpec((D, D), lambda t: (0, 0)),
            pl.BlockSpec((D, D), lambda t: (0, 0)),
            pl.BlockSpec((D, D), lambda t: (0, 0)),
            pl.BlockSpec((D, D), lambda t: (0, 0)),
        ],
        out_specs=pl.BlockSpec((BD, N),
                               lambda t: (jnp.maximum(t - T_DEC, 0), 0)),
        out_shape=jax.ShapeDtypeStruct((N, N), jnp.float32),
        scratch_shapes=[
            pltpu.VMEM((N, N), jnp.bfloat16),
            pltpu.VMEM((N, 64), jnp.bfloat16),
            pltpu.VMEM((N, 64), jnp.float32),
            pltpu.VMEM((N, K), jnp.bfloat16),
            pltpu.VMEM((K, 64), jnp.float32),
            pltpu.VMEM((K, 1), jnp.float32),
        ],
        compiler_params=pltpu.CompilerParams(
            vmem_limit_bytes=63 * 1024 * 1024),
    )(adj, C, x_bf, W1, Wp, Wc, Wb)

    return (mu.reshape(B, N, N), x)


# speedup vs baseline: 1.0681x; 1.0004x over previous
import jax
import jax.numpy as jnp
from jax import lax
from jax.experimental import pallas as pl
from jax.experimental.pallas import tpu as pltpu

N = 4096
BM = 256
NB = N // BM
BP = 2048
NP = N // BP
BD = 256
ND = N // BD
T_DEC = NB + NP


def _fused_kernel(adj_ref, c_ref, x_ref, w1_ref, wp_ref, wc_ref, wb_ref,
                  mu_ref, adj_bf, x1_bf, y_acc, c_bf, hb_s, cx_s, colsum_s):
    t = pl.program_id(0)

    @pl.when(t < NB)
    def _phase0():
        i = t
        a = adj_ref[...]
        a_bf = a.astype(jnp.bfloat16)
        adj_bf[pl.ds(i * BM, BM), :] = a_bf

        y = jnp.dot(a_bf, x_ref[...], preferred_element_type=jnp.float32)
        x1t = jnp.maximum(
            jnp.dot(y, w1_ref[...], preferred_element_type=jnp.float32), 0.0)
        x1_bf[pl.ds(i * BM, BM), :] = x1t.astype(jnp.bfloat16)

        c = c_ref[...]
        c_bf_blk = c.astype(jnp.bfloat16)
        c_bf[pl.ds(i * BM, BM), :] = c_bf_blk
        xc = x_ref[pl.ds(i * BM, BM), :]
        cx = lax.dot_general(c_bf_blk, xc, (((0,), (0,)), ((), ())),
                             preferred_element_type=jnp.float32)
        ones = jnp.ones((BM, 1), jnp.float32)
        cs = lax.dot_general(c, ones, (((0,), (0,)), ((), ())),
                             preferred_element_type=jnp.float32)

        @pl.when(t == 0)
        def _init():
            cx_s[...] = cx
            colsum_s[...] = cs

        @pl.when(t > 0)
        def _acc():
            cx_s[...] += cx
            colsum_s[...] += cs

    @pl.when(t == NB)
    def _cluster_finish():
        h1 = jnp.maximum(jnp.dot(cx_s[...], wp_ref[...],
                                 preferred_element_type=jnp.float32), 0.0)
        g = jnp.dot(h1, wb_ref[...], preferred_element_type=jnp.float32)
        gs = (g / colsum_s[...]).astype(jnp.bfloat16)
        hb_s[...] = jnp.dot(c_bf[...], gs, preferred_element_type=jnp.float32)

    @pl.when((t >= NB) & (t < T_DEC))
    def _phase1():
        i = t - NB
        a_bf = adj_bf[pl.ds(i * BP, BP), :]
        y = jnp.dot(a_bf, x1_bf[...], preferred_element_type=jnp.float32)
        y_acc[pl.ds(i * BP, BP), :] = jnp.maximum(
            jnp.dot(y, wc_ref[...], preferred_element_type=jnp.float32)
            + hb_s[pl.ds(i * BP, BP), :], 0.0)

    @pl.when(t >= T_DEC)
    def _phase2():
        i = t - T_DEC
        zb = y_acc[pl.ds(i * BD, BD), :]
        mu_ref[...] = jnp.maximum(
            lax.dot_general(zb, y_acc[...], (((1,), (1,)), ((), ())),
                            preferred_element_type=jnp.float32), 0.0)


def kernel(x, adj, C, W1, Wp, Wc, Wb):
    B, n, D = x.shape
    K = C.shape[1]
    x_bf = x[0].astype(jnp.bfloat16)

    mu = pl.pallas_call(
        _fused_kernel,
        grid=(NB + NP + ND,),
        in_specs=[
            pl.BlockSpec((BM, N), lambda t: (jnp.minimum(t, NB - 1), 0)),
            pl.BlockSpec((BM, K), lambda t: (jnp.minimum(t, NB - 1), 0)),
            pl.BlockSpec((N, D), lambda t: (0, 0)),
            pl.BlockSpec((D, D), lambda t: (0, 0)),
            pl.BlockSpec((D, D), lambda t: (0, 0)),
            pl.BlockSpec((D, D), lambda t: (0, 0)),
            pl.BlockSpec((D, D), lambda t: (0, 0)),
        ],
        out_specs=pl.BlockSpec((BD, N),
                               lambda t: (jnp.maximum(t - T_DEC, 0), 0)),
        out_shape=jax.ShapeDtypeStruct((N, N), jnp.float32),
        scratch_shapes=[
            pltpu.VMEM((N, N), jnp.bfloat16),
            pltpu.VMEM((N, 64), jnp.bfloat16),
            pltpu.VMEM((N, 64), jnp.float32),
            pltpu.VMEM((N, K), jnp.bfloat16),
            pltpu.VMEM((N, 64), jnp.float32),
            pltpu.VMEM((K, 64), jnp.float32),
            pltpu.VMEM((K, 1), jnp.float32),
        ],
        compiler_params=pltpu.CompilerParams(
            vmem_limit_bytes=63 * 1024 * 1024),
    )(adj, C, x_bf, W1, Wp, Wc, Wb)

    return (mu.reshape(B, N, N), x)
